# baseline (speedup 1.0000x reference)
import jax
import jax.numpy as jnp
from jax.experimental import pallas as pl
from jax.experimental.pallas import tpu as pltpu


_BM = 512


def _mlp_block(x_ref, w1_hbm, b1_ref, w2_hbm, b2_ref, o_ref,
               w1_scr, w2_scr, w1_sem, w2_sem):
    i = pl.program_id(0)
    w1_copy = pltpu.make_async_copy(w1_hbm, w1_scr, w1_sem)
    w2_copy = pltpu.make_async_copy(w2_hbm, w2_scr, w2_sem)

    @pl.when(i == 0)
    def _start():
        w1_copy.start()
        w2_copy.start()
        w1_copy.wait()

    xb = x_ref[...].astype(jnp.bfloat16)
    h = jax.lax.dot_general(
        xb, w1_scr[...], (((1,), (1,)), ((), ())),
        preferred_element_type=jnp.float32)
    h = jnp.maximum(h + b1_ref[...], 0.0)

    @pl.when(i == 0)
    def _wait_w2():
        w2_copy.wait()

    o = jax.lax.dot_general(
        h.astype(jnp.bfloat16), w2_scr[...], (((1,), (1,)), ((), ())),
        preferred_element_type=jnp.float32)
    o_ref[...] = o + b2_ref[...]


def kernel(input, W1, b1, W2, b2):
    M, K = input.shape
    N1, _ = W1.shape
    N2, _ = W2.shape
    return pl.pallas_call(
        _mlp_block,
        grid=(M // _BM,),
        in_specs=[
            pl.BlockSpec((_BM, K), lambda i: (i, 0)),
            pl.BlockSpec(memory_space=pl.ANY),
            pl.BlockSpec((1, N1), lambda i: (0, 0)),
            pl.BlockSpec(memory_space=pl.ANY),
            pl.BlockSpec((1, N2), lambda i: (0, 0)),
        ],
        out_specs=pl.BlockSpec((_BM, N2), lambda i: (i, 0)),
        out_shape=jax.ShapeDtypeStruct((M, N2), jnp.float32),
        scratch_shapes=[
            pltpu.VMEM((N1, K), jnp.float32),
            pltpu.VMEM((N2, N1), jnp.float32),
            pltpu.SemaphoreType.DMA,
            pltpu.SemaphoreType.DMA,
        ],
        compiler_params=pltpu.CompilerParams(
            vmem_limit_bytes=63 * 1024 * 1024),
    )(input, W1, b1.reshape(1, N1), W2, b2.reshape(1, N2))
